# baseline (speedup 1.0000x reference)
import jax
import jax.numpy as jnp
import numpy as np
from jax.experimental import pallas as pl
from jax.experimental.pallas import tpu as pltpu

_DIM = 64
_HALF = _DIM // 2
_MASK_TOKEN_ID = -10.0
_BLOCK_B = 128
_BLOCK_S = 40

_INV_PI = np.float32(1.0 / np.pi)
_PI = np.float32(np.pi)
_MAGIC = np.float32(12582912.0)
_C0 = np.float32(9.999994144953e-01)
_C1 = np.float32(-1.666583114777e-01)
_C2 = np.float32(8.315081746761e-03)
_C3 = np.float32(-1.857835029913e-04)


def _fast_sin(t):
    k = jax.lax.round(t * _INV_PI, jax.lax.RoundingMethod.TO_NEAREST_EVEN)
    r = t - k * _PI
    r2 = r * r
    p = r * (_C0 + r2 * (_C1 + r2 * (_C2 + r2 * _C3)))
    ki = k.astype(jnp.int32)
    sign = jax.lax.shift_left(ki, 31)
    return jax.lax.bitcast_convert_type(
        jax.lax.bitcast_convert_type(p, jnp.int32) ^ sign, jnp.float32
    )


def _rope_body(x0_ref, x1_ref, f_ref, p_ref, o_ref):
    bb2, s = x0_ref.shape
    lanes = 2 * _DIM
    x0 = jnp.broadcast_to(x0_ref[...][:, :, None], (bb2, s, lanes))
    x1 = jnp.broadcast_to(x1_ref[...][:, :, None], (bb2, s, lanes))
    lane = jax.lax.broadcasted_iota(jnp.int32, (bb2, s, lanes), 2)
    xb = jnp.where(lane < _DIM, x0, x1)
    angle = xb * f_ref[...] + p_ref[...]
    out = _fast_sin(angle)
    out = jnp.where(xb == _MASK_TOKEN_ID, jnp.float32(0.0), out)
    o_ref[0:bb2] = jax.lax.slice_in_dim(out, 0, _DIM, axis=2)
    o_ref[bb2 : 2 * bb2] = jax.lax.slice_in_dim(out, _DIM, lanes, axis=2)


def kernel(x, inv_freq):
    b, s = x.shape
    lanes = 2 * _DIM
    bb2 = _BLOCK_B // 2
    nb = b // _BLOCK_B

    ns = s // _BLOCK_S
    x4 = x.reshape(nb, 2, bb2, s)
    x0 = (
        x4[:, 0].reshape(nb * bb2, ns, _BLOCK_S)
        .swapaxes(0, 1).reshape(ns * nb * bb2, _BLOCK_S)
    )
    x1 = (
        x4[:, 1].reshape(nb * bb2, ns, _BLOCK_S)
        .swapaxes(0, 1).reshape(ns * nb * bb2, _BLOCK_S)
    )
    freq = jnp.tile(inv_freq, lanes // _HALF).reshape(1, 1, lanes)
    j = np.arange(lanes)
    phase = jnp.asarray(
        np.where((j % _DIM) >= _HALF, np.float32(np.pi / 2), np.float32(0.0)),
        dtype=jnp.float32,
    ).reshape(1, 1, lanes)

    return pl.pallas_call(
        _rope_body,
        grid=(nb, ns),
        in_specs=[
            pl.BlockSpec((bb2, _BLOCK_S), lambda i, k: (k * nb + i, 0)),
            pl.BlockSpec((bb2, _BLOCK_S), lambda i, k: (k * nb + i, 0)),
            pl.BlockSpec((1, 1, lanes), lambda i, k: (0, 0, 0)),
            pl.BlockSpec((1, 1, lanes), lambda i, k: (0, 0, 0)),
        ],
        out_specs=pl.BlockSpec((_BLOCK_B, _BLOCK_S, _DIM), lambda i, k: (i, k, 0)),
        out_shape=jax.ShapeDtypeStruct((b, s, _DIM), jnp.float32),
        compiler_params=pltpu.CompilerParams(
            dimension_semantics=("parallel", "parallel"),
        ),
    )(x0, x1, freq, phase)


# speedup vs baseline: 1.0211x; 1.0211x over previous
import jax
import jax.numpy as jnp
import numpy as np
from jax.experimental import pallas as pl
from jax.experimental.pallas import tpu as pltpu

_DIM = 64
_HALF = _DIM // 2
_MASK_TOKEN_ID = -10.0
_BLOCK_B = 128

_INV_PI = np.float32(1.0 / np.pi)
_PI = np.float32(np.pi)
_MAGIC = np.float32(12582912.0)
_C0 = np.float32(9.999994144953e-01)
_C1 = np.float32(-1.666583114777e-01)
_C2 = np.float32(8.315081746761e-03)
_C3 = np.float32(-1.857835029913e-04)


def _fast_sin(t):
    k = jax.lax.round(t * _INV_PI, jax.lax.RoundingMethod.TO_NEAREST_EVEN)
    r = t - k * _PI
    r2 = r * r
    p = r * (_C0 + r2 * (_C1 + r2 * (_C2 + r2 * _C3)))
    ki = k.astype(jnp.int32)
    sign = jax.lax.shift_left(ki, 31)
    return jax.lax.bitcast_convert_type(
        jax.lax.bitcast_convert_type(p, jnp.int32) ^ sign, jnp.float32
    )


_CHUNK = 8


def _rope_body(x0_ref, x1_ref, f_ref, p_ref, o_ref):
    bb2, s = x0_ref.shape
    lanes = 2 * _DIM
    fv = f_ref[...]
    pv = p_ref[...]

    def step(c, carry):
        base = c * _CHUNK
        x0 = jnp.broadcast_to(
            x0_ref[pl.ds(base, _CHUNK), :][:, :, None], (_CHUNK, s, lanes)
        )
        x1 = jnp.broadcast_to(
            x1_ref[pl.ds(base, _CHUNK), :][:, :, None], (_CHUNK, s, lanes)
        )
        lane = jax.lax.broadcasted_iota(jnp.int32, (_CHUNK, s, lanes), 2)
        xb = jnp.where(lane < _DIM, x0, x1)
        angle = xb * fv + pv
        out = _fast_sin(angle)
        out = jnp.where(xb == _MASK_TOKEN_ID, jnp.float32(0.0), out)
        o_ref[pl.ds(base, _CHUNK)] = jax.lax.slice_in_dim(out, 0, _DIM, axis=2)
        o_ref[pl.ds(bb2 + base, _CHUNK)] = jax.lax.slice_in_dim(
            out, _DIM, lanes, axis=2
        )
        return carry

    jax.lax.fori_loop(0, bb2 // _CHUNK, step, 0)


def kernel(x, inv_freq):
    b, s = x.shape
    lanes = 2 * _DIM
    bb2 = _BLOCK_B // 2
    nb = b // _BLOCK_B

    x4 = x.reshape(nb, 2, bb2, s)
    x0 = x4[:, 0].reshape(nb * bb2, s)
    x1 = x4[:, 1].reshape(nb * bb2, s)
    freq = jnp.tile(inv_freq, lanes // _HALF).reshape(1, 1, lanes)
    j = np.arange(lanes)
    phase = jnp.asarray(
        np.where((j % _DIM) >= _HALF, np.float32(np.pi / 2), np.float32(0.0)),
        dtype=jnp.float32,
    ).reshape(1, 1, lanes)

    return pl.pallas_call(
        _rope_body,
        grid=(nb,),
        in_specs=[
            pl.BlockSpec((bb2, s), lambda i: (i, 0)),
            pl.BlockSpec((bb2, s), lambda i: (i, 0)),
            pl.BlockSpec((1, 1, lanes), lambda i: (0, 0, 0)),
            pl.BlockSpec((1, 1, lanes), lambda i: (0, 0, 0)),
        ],
        out_specs=pl.BlockSpec((_BLOCK_B, s, _DIM), lambda i: (i, 0, 0)),
        out_shape=jax.ShapeDtypeStruct((b, s, _DIM), jnp.float32),
        compiler_params=pltpu.CompilerParams(
            dimension_semantics=("parallel",),
        ),
    )(x0, x1, freq, phase)


# speedup vs baseline: 1.0535x; 1.0318x over previous
import jax
import jax.numpy as jnp
import numpy as np
from jax.experimental import pallas as pl
from jax.experimental.pallas import tpu as pltpu

_DIM = 64
_HALF = _DIM // 2
_MASK_TOKEN_ID = -10.0
_BLOCK_B = 128

_INV_PI = np.float32(1.0 / np.pi)
_PI = np.float32(np.pi)
_MAGIC = np.float32(12582912.0)
_C0 = np.float32(9.999994144953e-01)
_C1 = np.float32(-1.666583114777e-01)
_C2 = np.float32(8.315081746761e-03)
_C3 = np.float32(-1.857835029913e-04)


def _fast_sin(t):
    k = jax.lax.round(t * _INV_PI, jax.lax.RoundingMethod.TO_NEAREST_EVEN)
    r = t - k * _PI
    r2 = r * r
    p = r * (_C0 + r2 * (_C1 + r2 * (_C2 + r2 * _C3)))
    ki = k.astype(jnp.int32)
    sign = jax.lax.shift_left(ki, 31)
    return jax.lax.bitcast_convert_type(
        jax.lax.bitcast_convert_type(p, jnp.int32) ^ sign, jnp.float32
    )


_CHUNK = 16


def _rope_body(x0_ref, x1_ref, f_ref, p_ref, o_ref):
    bb2, s = x0_ref.shape
    lanes = 2 * _DIM
    fv = f_ref[...]
    pv = p_ref[...]

    def step(c, carry):
        base = c * _CHUNK
        x0 = jnp.broadcast_to(
            x0_ref[pl.ds(base, _CHUNK), :][:, :, None], (_CHUNK, s, lanes)
        )
        x1 = jnp.broadcast_to(
            x1_ref[pl.ds(base, _CHUNK), :][:, :, None], (_CHUNK, s, lanes)
        )
        lane = jax.lax.broadcasted_iota(jnp.int32, (_CHUNK, s, lanes), 2)
        xb = jnp.where(lane < _DIM, x0, x1)
        angle = xb * fv + pv
        out = _fast_sin(angle)
        out = jnp.where(xb == _MASK_TOKEN_ID, jnp.float32(0.0), out)
        o_ref[pl.ds(base, _CHUNK)] = jax.lax.slice_in_dim(out, 0, _DIM, axis=2)
        o_ref[pl.ds(bb2 + base, _CHUNK)] = jax.lax.slice_in_dim(
            out, _DIM, lanes, axis=2
        )
        return carry

    jax.lax.fori_loop(0, bb2 // _CHUNK, step, 0)


def kernel(x, inv_freq):
    b, s = x.shape
    lanes = 2 * _DIM
    bb2 = _BLOCK_B // 2
    nb = b // _BLOCK_B

    x4 = x.reshape(nb, 2, bb2, s)
    x0 = x4[:, 0].reshape(nb * bb2, s)
    x1 = x4[:, 1].reshape(nb * bb2, s)
    freq = jnp.tile(inv_freq, lanes // _HALF).reshape(1, 1, lanes)
    j = np.arange(lanes)
    phase = jnp.asarray(
        np.where((j % _DIM) >= _HALF, np.float32(np.pi / 2), np.float32(0.0)),
        dtype=jnp.float32,
    ).reshape(1, 1, lanes)

    return pl.pallas_call(
        _rope_body,
        grid=(nb,),
        in_specs=[
            pl.BlockSpec((bb2, s), lambda i: (i, 0)),
            pl.BlockSpec((bb2, s), lambda i: (i, 0)),
            pl.BlockSpec((1, 1, lanes), lambda i: (0, 0, 0)),
            pl.BlockSpec((1, 1, lanes), lambda i: (0, 0, 0)),
        ],
        out_specs=pl.BlockSpec((_BLOCK_B, s, _DIM), lambda i: (i, 0, 0)),
        out_shape=jax.ShapeDtypeStruct((b, s, _DIM), jnp.float32),
        compiler_params=pltpu.CompilerParams(
            dimension_semantics=("parallel",),
        ),
    )(x0, x1, freq, phase)


# speedup vs baseline: 1.1095x; 1.0531x over previous
import jax
import jax.numpy as jnp
import numpy as np
from jax.experimental import pallas as pl
from jax.experimental.pallas import tpu as pltpu

_DIM = 64
_HALF = _DIM // 2
_MASK_TOKEN_ID = -10.0
_BLOCK_B = 128

_C0 = np.float32(6.283168279489e00)
_C1 = np.float32(-4.133793036373e01)
_C2 = np.float32(8.147314577201e01)
_C3 = np.float32(-7.509336954871e01)
_C4 = np.float32(3.395672172413e01)


def _sin_turns(u):
    k = jax.lax.round(u, jax.lax.RoundingMethod.TO_NEAREST_EVEN)
    d = u - k
    d2 = d * d
    return d * (_C0 + d2 * (_C1 + d2 * (_C2 + d2 * (_C3 + d2 * _C4))))


def _rope_body(x0_ref, x1_ref, g_ref, q_ref, o_ref):
    bb2, s = x0_ref.shape
    lanes = 2 * _DIM
    x0 = jnp.broadcast_to(x0_ref[...][:, :, None], (bb2, s, lanes))
    x1 = jnp.broadcast_to(x1_ref[...][:, :, None], (bb2, s, lanes))
    lane = jax.lax.broadcasted_iota(jnp.int32, (bb2, s, lanes), 2)
    xb = jnp.where(lane < _DIM, x0, x1)
    out = _sin_turns(xb * g_ref[...] + q_ref[...])
    out = jnp.where(xb == _MASK_TOKEN_ID, jnp.float32(0.0), out)
    o_ref[0:bb2] = jax.lax.slice_in_dim(out, 0, _DIM, axis=2)
    o_ref[bb2 : 2 * bb2] = jax.lax.slice_in_dim(out, _DIM, lanes, axis=2)


def kernel(x, inv_freq):
    b, s = x.shape
    lanes = 2 * _DIM
    bb2 = _BLOCK_B // 2
    nb = b // _BLOCK_B

    x4 = x.reshape(nb, 2, bb2, s)
    x0 = x4[:, 0].reshape(nb * bb2, s)
    x1 = x4[:, 1].reshape(nb * bb2, s)

    inv2pi = np.float32(1.0 / (2.0 * np.pi))
    g = (jnp.tile(inv_freq, lanes // _HALF) * inv2pi).reshape(1, 1, lanes)
    j = np.arange(lanes)
    q = jnp.asarray(
        np.where((j % _DIM) >= _HALF, np.float32(0.25), np.float32(0.0)),
        dtype=jnp.float32,
    ).reshape(1, 1, lanes)

    return pl.pallas_call(
        _rope_body,
        grid=(nb,),
        in_specs=[
            pl.BlockSpec((bb2, s), lambda i: (i, 0)),
            pl.BlockSpec((bb2, s), lambda i: (i, 0)),
            pl.BlockSpec((1, 1, lanes), lambda i: (0, 0, 0)),
            pl.BlockSpec((1, 1, lanes), lambda i: (0, 0, 0)),
        ],
        out_specs=pl.BlockSpec((_BLOCK_B, s, _DIM), lambda i: (i, 0, 0)),
        out_shape=jax.ShapeDtypeStruct((b, s, _DIM), jnp.float32),
        compiler_params=pltpu.CompilerParams(
            dimension_semantics=("parallel",),
        ),
    )(x0, x1, g, q)


# speedup vs baseline: 1.1299x; 1.0184x over previous
import jax
import jax.numpy as jnp
import numpy as np
from jax.experimental import pallas as pl
from jax.experimental.pallas import tpu as pltpu

_DIM = 64
_HALF = _DIM // 2
_MASK_TOKEN_ID = -10.0
_BLOCK_B = 128

_C0 = np.float32(6.282446802e00)
_C1 = np.float32(-4.123403555e01)
_C2 = np.float32(7.918749660e01)
_C3 = np.float32(-5.924653714e01)


def _sin_turns(u):
    k = jax.lax.round(u, jax.lax.RoundingMethod.TO_NEAREST_EVEN)
    d = u - k
    d2 = d * d
    return d * (_C0 + d2 * (_C1 + d2 * (_C2 + d2 * _C3)))


def _rope_body(x0_ref, x1_ref, g_ref, q_ref, o_ref):
    bb2, s = x0_ref.shape
    lanes = 2 * _DIM
    x0 = jnp.broadcast_to(x0_ref[...][:, :, None], (bb2, s, lanes))
    x1 = jnp.broadcast_to(x1_ref[...][:, :, None], (bb2, s, lanes))
    lane = jax.lax.broadcasted_iota(jnp.int32, (bb2, s, lanes), 2)
    xb = jnp.where(lane < _DIM, x0, x1)
    out = _sin_turns(xb * g_ref[...] + q_ref[...])
    out = jnp.where(xb == _MASK_TOKEN_ID, jnp.float32(0.0), out)
    o_ref[0:bb2] = jax.lax.slice_in_dim(out, 0, _DIM, axis=2)
    o_ref[bb2 : 2 * bb2] = jax.lax.slice_in_dim(out, _DIM, lanes, axis=2)


def kernel(x, inv_freq):
    b, s = x.shape
    lanes = 2 * _DIM
    bb2 = _BLOCK_B // 2
    nb = b // _BLOCK_B

    x4 = x.reshape(nb, 2, bb2, s)
    x0 = x4[:, 0].reshape(nb * bb2, s)
    x1 = x4[:, 1].reshape(nb * bb2, s)

    inv2pi = np.float32(1.0 / (2.0 * np.pi))
    g = (jnp.tile(inv_freq, lanes // _HALF) * inv2pi).reshape(1, 1, lanes)
    j = np.arange(lanes)
    q = jnp.asarray(
        np.where((j % _DIM) >= _HALF, np.float32(0.25), np.float32(0.0)),
        dtype=jnp.float32,
    ).reshape(1, 1, lanes)

    return pl.pallas_call(
        _rope_body,
        grid=(nb,),
        in_specs=[
            pl.BlockSpec((bb2, s), lambda i: (i, 0)),
            pl.BlockSpec((bb2, s), lambda i: (i, 0)),
            pl.BlockSpec((1, 1, lanes), lambda i: (0, 0, 0)),
            pl.BlockSpec((1, 1, lanes), lambda i: (0, 0, 0)),
        ],
        out_specs=pl.BlockSpec((_BLOCK_B, s, _DIM), lambda i: (i, 0, 0)),
        out_shape=jax.ShapeDtypeStruct((b, s, _DIM), jnp.float32),
        compiler_params=pltpu.CompilerParams(
            dimension_semantics=("parallel",),
        ),
    )(x0, x1, g, q)


# speedup vs baseline: 1.1443x; 1.0127x over previous
import jax
import jax.numpy as jnp
import numpy as np
from jax.experimental import pallas as pl
from jax.experimental.pallas import tpu as pltpu

_DIM = 64
_HALF = _DIM // 2
_MASK_TOKEN_ID = -10.0
_BLOCK_B = 64

_C0 = np.float32(6.282446802e00)
_C1 = np.float32(-4.123403555e01)
_C2 = np.float32(7.918749660e01)
_C3 = np.float32(-5.924653714e01)


def _sin_turns(u):
    k = jax.lax.round(u, jax.lax.RoundingMethod.TO_NEAREST_EVEN)
    d = u - k
    d2 = d * d
    return d * (_C0 + d2 * (_C1 + d2 * (_C2 + d2 * _C3)))


def _rope_body(x0_ref, x1_ref, g_ref, q_ref, o_ref):
    bb2, s = x0_ref.shape
    lanes = 2 * _DIM
    x0 = jnp.broadcast_to(x0_ref[...][:, :, None], (bb2, s, lanes))
    x1 = jnp.broadcast_to(x1_ref[...][:, :, None], (bb2, s, lanes))
    lane = jax.lax.broadcasted_iota(jnp.int32, (bb2, s, lanes), 2)
    xb = jnp.where(lane < _DIM, x0, x1)
    masked = xb == _MASK_TOKEN_ID
    out = _sin_turns(xb * g_ref[...] + q_ref[...])
    out = jnp.where(masked, jnp.float32(0.0), out)
    o_ref[0:bb2] = jax.lax.slice_in_dim(out, 0, _DIM, axis=2)
    o_ref[bb2 : 2 * bb2] = jax.lax.slice_in_dim(out, _DIM, lanes, axis=2)


def kernel(x, inv_freq):
    b, s = x.shape
    lanes = 2 * _DIM
    bb2 = _BLOCK_B // 2
    nb = b // _BLOCK_B

    x4 = x.reshape(nb, 2, bb2, s)
    x0 = x4[:, 0].reshape(nb * bb2, s)
    x1 = x4[:, 1].reshape(nb * bb2, s)

    inv2pi = np.float32(1.0 / (2.0 * np.pi))
    g = (jnp.tile(inv_freq, lanes // _HALF) * inv2pi).reshape(1, 1, lanes)
    j = np.arange(lanes)
    q = jnp.asarray(
        np.where((j % _DIM) >= _HALF, np.float32(0.25), np.float32(0.0)),
        dtype=jnp.float32,
    ).reshape(1, 1, lanes)

    return pl.pallas_call(
        _rope_body,
        grid=(nb,),
        in_specs=[
            pl.BlockSpec((bb2, s), lambda i: (i, 0)),
            pl.BlockSpec((bb2, s), lambda i: (i, 0)),
            pl.BlockSpec((1, 1, lanes), lambda i: (0, 0, 0)),
            pl.BlockSpec((1, 1, lanes), lambda i: (0, 0, 0)),
        ],
        out_specs=pl.BlockSpec((_BLOCK_B, s, _DIM), lambda i: (i, 0, 0)),
        out_shape=jax.ShapeDtypeStruct((b, s, _DIM), jnp.float32),
        compiler_params=pltpu.CompilerParams(
            dimension_semantics=("parallel",),
        ),
    )(x0, x1, g, q)
